# Initial kernel scaffold; baseline (speedup 1.0000x reference)
import jax
import jax.numpy as jnp
from jax.experimental import pallas as pl


def kernel(x, mask, ln1_scale, ln1_bias, ln2_scale, ln2_bias, W_q, W_k, W_v, W_o, router_W, gate_W, W1, W2):
    raise NotImplementedError("write your pallas kernel here")


# speedup vs baseline: 4.3049x; 4.3049x over previous
import math

import jax
import jax.numpy as jnp
from jax.experimental import pallas as pl
from jax.experimental.pallas import tpu as pltpu

S, D, H, E, K, DFF = 2048, 768, 12, 8, 2, 3072
DH = D // H
CAP = math.ceil(1.25 * S / E)
RB = 256
NRB = S // RB
F32 = jnp.float32


def _ln(x, scale, bias):
    n = x.shape[0]
    xr = x.reshape(n, D // 128, 128)
    red = lambda v: jnp.sum(jnp.sum(v.reshape(n, D // 128, 128), axis=2),
                            axis=-1, keepdims=True)
    mu = red(x) / D
    var = red((x - mu) ** 2) / D
    return (x - mu) / jnp.sqrt(var + 1e-5) * scale + bias


def _prep1_body(x_ref, s_ref, b_ref, wq_ref, wk_ref, wr_ref,
                xn_ref, q_ref, k_ref, gl_ref):
    xn = _ln(x_ref[...], s_ref[...], b_ref[...])
    xn_ref[...] = xn
    q_ref[...] = jnp.dot(xn, wq_ref[...], preferred_element_type=F32)
    k_ref[...] = jnp.dot(xn, wk_ref[...], preferred_element_type=F32)
    gl_ref[...] = jnp.dot(xn, wr_ref[...], preferred_element_type=F32)


def _prep1(x, ln_s, ln_b, wq, wk, wr):
    return pl.pallas_call(
        _prep1_body,
        grid=(NRB,),
        in_specs=[
            pl.BlockSpec((RB, D), lambda i: (i, 0)),
            pl.BlockSpec((1, D), lambda i: (0, 0)),
            pl.BlockSpec((1, D), lambda i: (0, 0)),
            pl.BlockSpec((D, D), lambda i: (0, 0)),
            pl.BlockSpec((D, D), lambda i: (0, 0)),
            pl.BlockSpec((D, H * E), lambda i: (0, 0)),
        ],
        out_specs=[
            pl.BlockSpec((RB, D), lambda i: (i, 0)),
            pl.BlockSpec((RB, D), lambda i: (i, 0)),
            pl.BlockSpec((RB, D), lambda i: (i, 0)),
            pl.BlockSpec((RB, H * E), lambda i: (i, 0)),
        ],
        out_shape=[
            jax.ShapeDtypeStruct((S, D), F32),
            jax.ShapeDtypeStruct((S, D), F32),
            jax.ShapeDtypeStruct((S, D), F32),
            jax.ShapeDtypeStruct((S, H * E), F32),
        ],
    )(x, ln_s, ln_b, wq, wk, wr)


def _top2(g):
    iota = jax.lax.broadcasted_iota(jnp.int32, g.shape, g.ndim - 1)
    m1 = jnp.max(g, axis=-1, keepdims=True)
    i1 = jnp.min(jnp.where(g == m1, iota, E), axis=-1, keepdims=True)
    oh1 = (iota == i1).astype(F32)
    g2 = jnp.where(oh1 > 0, -jnp.inf, g)
    m2 = jnp.max(g2, axis=-1, keepdims=True)
    i2 = jnp.min(jnp.where(g2 == m2, iota, E), axis=-1, keepdims=True)
    oh2 = (iota == i2).astype(F32)
    e2 = jnp.exp(m2 - m1)
    p1 = 1.0 / (1.0 + e2)
    p2 = e2 / (1.0 + e2)
    return oh1, oh2, p1, p2


def _route_body(gl_ref, oh1_ref, oh2_ref, php1_ref, php2_ref, hist_ref):
    g = gl_ref[...]
    oh1, oh2, p1, p2 = _top2(g)
    oh1_ref[...] = oh1
    oh2_ref[...] = oh2
    php1_ref[...] = oh1 * p1
    php2_ref[...] = oh2 * p2
    part = jnp.sum(oh1 + oh2, axis=0)

    @pl.when(pl.program_id(0) == 0)
    def _():
        hist_ref[...] = jnp.zeros_like(hist_ref)

    hist_ref[...] += part


def _route(gl3):
    return pl.pallas_call(
        _route_body,
        grid=(NRB,),
        in_specs=[pl.BlockSpec((RB, H, E), lambda i: (i, 0, 0))],
        out_specs=[
            pl.BlockSpec((RB, H, E), lambda i: (i, 0, 0)),
            pl.BlockSpec((RB, H, E), lambda i: (i, 0, 0)),
            pl.BlockSpec((RB, H, E), lambda i: (i, 0, 0)),
            pl.BlockSpec((RB, H, E), lambda i: (i, 0, 0)),
            pl.BlockSpec((H, E), lambda i: (0, 0)),
        ],
        out_shape=[
            jax.ShapeDtypeStruct((S, H, E), F32),
            jax.ShapeDtypeStruct((S, H, E), F32),
            jax.ShapeDtypeStruct((S, H, E), F32),
            jax.ShapeDtypeStruct((S, H, E), F32),
            jax.ShapeDtypeStruct((H, E), F32),
        ],
    )(gl3)


def _vproj_body(xh_ref, wv_ref, oh1_ref, oh2_ref, v1_ref, v2_ref):
    x = xh_ref[0]
    oh1 = oh1_ref[0]
    oh2 = oh2_ref[0]
    acc1 = jnp.zeros((S, DH), F32)
    acc2 = jnp.zeros((S, DH), F32)
    for e in range(E):
        proj = jnp.dot(x, wv_ref[0, e], preferred_element_type=F32)
        acc1 = acc1 + oh1[:, e:e + 1] * proj
        acc2 = acc2 + oh2[:, e:e + 1] * proj
    v1_ref[0] = acc1
    v2_ref[0] = acc2


def _vproj(xh, wv, oh1t, oh2t):
    return pl.pallas_call(
        _vproj_body,
        grid=(H,),
        in_specs=[
            pl.BlockSpec((1, S, DH), lambda h: (h, 0, 0)),
            pl.BlockSpec((1, E, DH, DH), lambda h: (h, 0, 0, 0)),
            pl.BlockSpec((1, S, E), lambda h: (h, 0, 0)),
            pl.BlockSpec((1, S, E), lambda h: (h, 0, 0)),
        ],
        out_specs=[
            pl.BlockSpec((1, S, DH), lambda h: (h, 0, 0)),
            pl.BlockSpec((1, S, DH), lambda h: (h, 0, 0)),
        ],
        out_shape=[
            jax.ShapeDtypeStruct((H, S, DH), F32),
            jax.ShapeDtypeStruct((H, S, DH), F32),
        ],
    )(xh, wv, oh1t, oh2t)


def _attn_body(q_ref, k_ref, v1_ref, v2_ref, m_ref, a1_ref, a2_ref):
    q = q_ref[0]
    k = k_ref[0]
    s = jax.lax.dot_general(q, k, (((1,), (1,)), ((), ())),
                            preferred_element_type=F32)
    s = s * (1.0 / math.sqrt(DH)) + m_ref[...]
    mx = jnp.max(s, axis=-1, keepdims=True)
    p = jnp.exp(s - mx)
    p = p / jnp.sum(p, axis=-1, keepdims=True)

    @pl.when(pl.program_id(1) == 0)
    def _():
        a1_ref[0] = jnp.zeros((S, DH), F32)
        a2_ref[0] = jnp.zeros((S, DH), F32)

    a1_ref[0] += jax.lax.dot_general(p, v1_ref[0], (((0,), (0,)), ((), ())),
                                     preferred_element_type=F32)
    a2_ref[0] += jax.lax.dot_general(p, v2_ref[0], (((0,), (0,)), ((), ())),
                                     preferred_element_type=F32)


def _attn(qh, kh, v1, v2, mask):
    return pl.pallas_call(
        _attn_body,
        grid=(H, NRB),
        in_specs=[
            pl.BlockSpec((1, RB, DH), lambda h, i: (h, i, 0)),
            pl.BlockSpec((1, S, DH), lambda h, i: (h, 0, 0)),
            pl.BlockSpec((1, RB, DH), lambda h, i: (h, i, 0)),
            pl.BlockSpec((1, RB, DH), lambda h, i: (h, i, 0)),
            pl.BlockSpec((RB, S), lambda h, i: (i, 0)),
        ],
        out_specs=[
            pl.BlockSpec((1, S, DH), lambda h, i: (h, 0, 0)),
            pl.BlockSpec((1, S, DH), lambda h, i: (h, 0, 0)),
        ],
        out_shape=[
            jax.ShapeDtypeStruct((H, S, DH), F32),
            jax.ShapeDtypeStruct((H, S, DH), F32),
        ],
    )(qh, kh, v1, v2, mask)


def _oproj_body(a1_ref, a2_ref, wo_ref, php1_ref, php2_ref, o_ref):
    a1 = a1_ref[0]
    a2 = a2_ref[0]
    php1 = php1_ref[0]
    php2 = php2_ref[0]
    acc = jnp.zeros((S, DH), F32)
    for e in range(E):
        c = php1[:, e:e + 1] * a1 + php2[:, e:e + 1] * a2
        acc = acc + jnp.dot(c, wo_ref[0, e], preferred_element_type=F32)
    o_ref[0] = acc


def _oproj(a1, a2, wo, php1t, php2t):
    return pl.pallas_call(
        _oproj_body,
        grid=(H,),
        in_specs=[
            pl.BlockSpec((1, S, DH), lambda h: (h, 0, 0)),
            pl.BlockSpec((1, S, DH), lambda h: (h, 0, 0)),
            pl.BlockSpec((1, E, DH, DH), lambda h: (h, 0, 0, 0)),
            pl.BlockSpec((1, S, E), lambda h: (h, 0, 0)),
            pl.BlockSpec((1, S, E), lambda h: (h, 0, 0)),
        ],
        out_specs=pl.BlockSpec((1, S, DH), lambda h: (h, 0, 0)),
        out_shape=jax.ShapeDtypeStruct((H, S, DH), F32),
    )(a1, a2, wo, php1t, php2t)


def _prep2_body(x_ref, ao_ref, s_ref, b_ref, gw_ref,
                xm_ref, xn2_ref, oh1_ref, oh2_ref, php1_ref, php2_ref):
    xm = x_ref[...] + ao_ref[...]
    xm_ref[...] = xm
    xn2 = _ln(xm, s_ref[...], b_ref[...])
    xn2_ref[...] = xn2
    gl = jnp.dot(xn2, gw_ref[...], preferred_element_type=F32)
    oh1, oh2, p1, p2 = _top2(gl)
    oh1_ref[...] = oh1
    oh2_ref[...] = oh2
    php1_ref[...] = oh1 * p1
    php2_ref[...] = oh2 * p2


def _prep2(x, ao, ln_s, ln_b, gw):
    return pl.pallas_call(
        _prep2_body,
        grid=(NRB,),
        in_specs=[
            pl.BlockSpec((RB, D), lambda i: (i, 0)),
            pl.BlockSpec((RB, D), lambda i: (i, 0)),
            pl.BlockSpec((1, D), lambda i: (0, 0)),
            pl.BlockSpec((1, D), lambda i: (0, 0)),
            pl.BlockSpec((D, E), lambda i: (0, 0)),
        ],
        out_specs=[
            pl.BlockSpec((RB, D), lambda i: (i, 0)),
            pl.BlockSpec((RB, D), lambda i: (i, 0)),
            pl.BlockSpec((RB, E), lambda i: (i, 0)),
            pl.BlockSpec((RB, E), lambda i: (i, 0)),
            pl.BlockSpec((RB, E), lambda i: (i, 0)),
            pl.BlockSpec((RB, E), lambda i: (i, 0)),
        ],
        out_shape=[
            jax.ShapeDtypeStruct((S, D), F32),
            jax.ShapeDtypeStruct((S, D), F32),
            jax.ShapeDtypeStruct((S, E), F32),
            jax.ShapeDtypeStruct((S, E), F32),
            jax.ShapeDtypeStruct((S, E), F32),
            jax.ShapeDtypeStruct((S, E), F32),
        ],
    )(x, ao, ln_s, ln_b, gw)


def _cap_body(oh1_ref, oh2_ref, php1_ref, php2_ref,
              w_ref, tok_ref, imp_ref, run_ref):
    i = pl.program_id(0)

    @pl.when(i == 0)
    def _():
        run_ref[...] = jnp.zeros_like(run_ref)
        tok_ref[...] = jnp.zeros_like(tok_ref)
        imp_ref[...] = jnp.zeros_like(imp_ref)

    oh1 = oh1_ref[...]
    oh2 = oh2_ref[...]
    ohsum = oh1 + oh2
    r = jax.lax.broadcasted_iota(jnp.int32, (RB, RB), 0)
    c = jax.lax.broadcasted_iota(jnp.int32, (RB, RB), 1)
    ltri = (r > c).astype(F32)
    before = jnp.dot(ltri, ohsum, preferred_element_type=F32) + run_ref[...]
    pos0 = before + 1.0
    pos1 = before + oh1 + 1.0
    keep0 = (pos0 <= CAP).astype(F32)
    keep1 = (pos1 <= CAP).astype(F32)
    pr0 = php1_ref[...] * keep0
    pr1 = php2_ref[...] * keep1
    denom = jnp.sum(pr0 + pr1, axis=-1, keepdims=True) + 1e-9
    pr0 = pr0 / denom
    pr1 = pr1 / denom
    w_ref[...] = pr0 + pr1
    m0 = jnp.where(pr0 > 0, oh1, 0.0)
    m1 = jnp.where(pr1 > 0, oh2, 0.0)
    tok_ref[...] += jnp.sum(m0 + m1, axis=0, keepdims=True)
    imp_ref[...] += jnp.sum(pr0 + pr1, axis=0, keepdims=True)
    run_ref[...] += jnp.sum(ohsum, axis=0, keepdims=True)


def _capacity(oh1, oh2, php1, php2):
    return pl.pallas_call(
        _cap_body,
        grid=(NRB,),
        in_specs=[
            pl.BlockSpec((RB, E), lambda i: (i, 0)),
            pl.BlockSpec((RB, E), lambda i: (i, 0)),
            pl.BlockSpec((RB, E), lambda i: (i, 0)),
            pl.BlockSpec((RB, E), lambda i: (i, 0)),
        ],
        out_specs=[
            pl.BlockSpec((RB, E), lambda i: (i, 0)),
            pl.BlockSpec((1, E), lambda i: (0, 0)),
            pl.BlockSpec((1, E), lambda i: (0, 0)),
        ],
        out_shape=[
            jax.ShapeDtypeStruct((S, E), F32),
            jax.ShapeDtypeStruct((1, E), F32),
            jax.ShapeDtypeStruct((1, E), F32),
        ],
        scratch_shapes=[pltpu.VMEM((1, E), F32)],
    )(oh1, oh2, php1, php2)


def _ffn_body(xn2_ref, w_ref, xm_ref, w1_ref, w2_ref, y_ref):
    e = pl.program_id(1)
    lane = jax.lax.broadcasted_iota(jnp.int32, (RB, E), 1)
    we = jnp.sum(jnp.where(lane == e, w_ref[...], 0.0), axis=-1, keepdims=True)
    xw = xn2_ref[...] * we
    h1 = jnp.maximum(jnp.dot(xw, w1_ref[0], preferred_element_type=F32), 0.0)
    out = jnp.dot(h1, w2_ref[0], preferred_element_type=F32)

    @pl.when(e == 0)
    def _():
        y_ref[...] = xm_ref[...]

    y_ref[...] += out


def _ffn(xn2, w, xm, w1, w2):
    return pl.pallas_call(
        _ffn_body,
        grid=(NRB, E),
        in_specs=[
            pl.BlockSpec((RB, D), lambda i, e: (i, 0)),
            pl.BlockSpec((RB, E), lambda i, e: (i, 0)),
            pl.BlockSpec((RB, D), lambda i, e: (i, 0)),
            pl.BlockSpec((1, D, DFF), lambda i, e: (e, 0, 0)),
            pl.BlockSpec((1, DFF, D), lambda i, e: (e, 0, 0)),
        ],
        out_specs=pl.BlockSpec((RB, D), lambda i, e: (i, 0)),
        out_shape=jax.ShapeDtypeStruct((S, D), F32),
    )(xn2, w, xm, w1, w2)


def _aux_body(hist_ref, tok_ref, imp_ref, o_ref):
    ema = 0.01 * hist_ref[...] / (S * K)
    pb = ema / (jnp.sum(ema) + 1e-9)
    aux1 = jnp.sum(pb * pb) * (E * H)
    tok = tok_ref[...]
    imp = imp_ref[...]
    aux2 = jnp.sum((tok / jnp.sum(tok)) * (imp / jnp.sum(imp))) * E
    o_ref[...] = jnp.full((1, 1), aux1 + aux2, F32)


def _aux(hist, tok, imp):
    return pl.pallas_call(
        _aux_body,
        in_specs=[
            pl.BlockSpec((H, E), lambda: (0, 0)),
            pl.BlockSpec((1, E), lambda: (0, 0)),
            pl.BlockSpec((1, E), lambda: (0, 0)),
        ],
        out_specs=pl.BlockSpec((1, 1), lambda: (0, 0)),
        out_shape=jax.ShapeDtypeStruct((1, 1), F32),
    )(hist, tok, imp)


def kernel(x, mask, ln1_scale, ln1_bias, ln2_scale, ln2_bias,
           W_q, W_k, W_v, W_o, router_W, gate_W, W1, W2):
    x2 = x.reshape(S, D)
    xn, q, k, gl = _prep1(x2, ln1_scale.reshape(1, D), ln1_bias.reshape(1, D),
                          W_q, W_k, router_W)
    oh1, oh2, php1, php2, hist = _route(gl.reshape(S, H, E))

    to_h = lambda a: a.reshape(S, H, DH).transpose(1, 0, 2)
    te = lambda a: a.transpose(1, 0, 2)
    v1, v2 = _vproj(to_h(xn), W_v, te(oh1), te(oh2))
    a1, a2 = _attn(to_h(q), to_h(k), v1, v2, mask)
    ao = _oproj(a1, a2, W_o, te(php1), te(php2))
    ao2 = ao.transpose(1, 0, 2).reshape(S, D)

    xm, xn2, goh1, goh2, gphp1, gphp2 = _prep2(
        x2, ao2, ln2_scale.reshape(1, D), ln2_bias.reshape(1, D), gate_W)
    w, tok, imp = _capacity(goh1, goh2, gphp1, gphp2)
    y = _ffn(xn2, w, xm, W1, W2)
    aux = _aux(hist, tok, imp)
    return y.reshape(1, S, D), aux.reshape(())


# speedup vs baseline: 5.1062x; 1.1861x over previous
import math

import jax
import jax.numpy as jnp
from jax.experimental import pallas as pl
from jax.experimental.pallas import tpu as pltpu

S, D, H, E, K, DFF = 2048, 768, 12, 8, 2, 3072
DH = D // H
CAP = math.ceil(1.25 * S / E)
RB = 256
NRB = S // RB
F32 = jnp.float32


def _ln(x, scale, bias):
    n = x.shape[0]
    xr = x.reshape(n, D // 128, 128)
    red = lambda v: jnp.sum(jnp.sum(v.reshape(n, D // 128, 128), axis=2),
                            axis=-1, keepdims=True)
    mu = red(x) / D
    var = red((x - mu) ** 2) / D
    return (x - mu) / jnp.sqrt(var + 1e-5) * scale + bias


def _prep1_body(x_ref, s_ref, b_ref, wq_ref, wk_ref, wr_ref,
                xn_ref, q_ref, k_ref, gl_ref):
    xn = _ln(x_ref[...], s_ref[...], b_ref[...])
    xn_ref[...] = xn
    q_ref[...] = jnp.dot(xn, wq_ref[...], preferred_element_type=F32)
    k_ref[...] = jnp.dot(xn, wk_ref[...], preferred_element_type=F32)
    gl_ref[...] = jnp.dot(xn, wr_ref[...], preferred_element_type=F32)


def _prep1(x, ln_s, ln_b, wq, wk, wr):
    return pl.pallas_call(
        _prep1_body,
        grid=(NRB,),
        in_specs=[
            pl.BlockSpec((RB, D), lambda i: (i, 0)),
            pl.BlockSpec((1, D), lambda i: (0, 0)),
            pl.BlockSpec((1, D), lambda i: (0, 0)),
            pl.BlockSpec((D, D), lambda i: (0, 0)),
            pl.BlockSpec((D, D), lambda i: (0, 0)),
            pl.BlockSpec((D, H * E), lambda i: (0, 0)),
        ],
        out_specs=[
            pl.BlockSpec((RB, D), lambda i: (i, 0)),
            pl.BlockSpec((RB, D), lambda i: (i, 0)),
            pl.BlockSpec((RB, D), lambda i: (i, 0)),
            pl.BlockSpec((RB, H * E), lambda i: (i, 0)),
        ],
        out_shape=[
            jax.ShapeDtypeStruct((S, D), F32),
            jax.ShapeDtypeStruct((S, D), F32),
            jax.ShapeDtypeStruct((S, D), F32),
            jax.ShapeDtypeStruct((S, H * E), F32),
        ],
    )(x, ln_s, ln_b, wq, wk, wr)


def _top2(g):
    iota = jax.lax.broadcasted_iota(jnp.int32, g.shape, g.ndim - 1)
    m1 = jnp.max(g, axis=-1, keepdims=True)
    i1 = jnp.min(jnp.where(g == m1, iota, E), axis=-1, keepdims=True)
    oh1 = (iota == i1).astype(F32)
    g2 = jnp.where(oh1 > 0, -jnp.inf, g)
    m2 = jnp.max(g2, axis=-1, keepdims=True)
    i2 = jnp.min(jnp.where(g2 == m2, iota, E), axis=-1, keepdims=True)
    oh2 = (iota == i2).astype(F32)
    e2 = jnp.exp(m2 - m1)
    p1 = 1.0 / (1.0 + e2)
    p2 = e2 / (1.0 + e2)
    return oh1, oh2, p1, p2


def _route_body(gl_ref, oh1_ref, oh2_ref, php1_ref, php2_ref, hist_ref):
    g = gl_ref[...]
    oh1, oh2, p1, p2 = _top2(g)
    oh1_ref[...] = oh1
    oh2_ref[...] = oh2
    php1_ref[...] = oh1 * p1
    php2_ref[...] = oh2 * p2
    part = jnp.sum(oh1 + oh2, axis=0)

    @pl.when(pl.program_id(0) == 0)
    def _():
        hist_ref[...] = jnp.zeros_like(hist_ref)

    hist_ref[...] += part


def _route(gl3):
    return pl.pallas_call(
        _route_body,
        grid=(NRB,),
        in_specs=[pl.BlockSpec((RB, H, E), lambda i: (i, 0, 0))],
        out_specs=[
            pl.BlockSpec((RB, H, E), lambda i: (i, 0, 0)),
            pl.BlockSpec((RB, H, E), lambda i: (i, 0, 0)),
            pl.BlockSpec((RB, H, E), lambda i: (i, 0, 0)),
            pl.BlockSpec((RB, H, E), lambda i: (i, 0, 0)),
            pl.BlockSpec((H, E), lambda i: (0, 0)),
        ],
        out_shape=[
            jax.ShapeDtypeStruct((S, H, E), F32),
            jax.ShapeDtypeStruct((S, H, E), F32),
            jax.ShapeDtypeStruct((S, H, E), F32),
            jax.ShapeDtypeStruct((S, H, E), F32),
            jax.ShapeDtypeStruct((H, E), F32),
        ],
    )(gl3)


def _vproj_body(xh_ref, wv_ref, oh1_ref, oh2_ref, v1_ref, v2_ref):
    x = xh_ref[0]
    oh1 = oh1_ref[0]
    oh2 = oh2_ref[0]
    acc1 = jnp.zeros((S, DH), F32)
    acc2 = jnp.zeros((S, DH), F32)
    for e in range(E):
        proj = jnp.dot(x, wv_ref[0, e], preferred_element_type=F32)
        acc1 = acc1 + oh1[:, e:e + 1] * proj
        acc2 = acc2 + oh2[:, e:e + 1] * proj
    v1_ref[0] = acc1
    v2_ref[0] = acc2


def _vproj(xh, wv, oh1t, oh2t):
    return pl.pallas_call(
        _vproj_body,
        grid=(H,),
        in_specs=[
            pl.BlockSpec((1, S, DH), lambda h: (h, 0, 0)),
            pl.BlockSpec((1, E, DH, DH), lambda h: (h, 0, 0, 0)),
            pl.BlockSpec((1, S, E), lambda h: (h, 0, 0)),
            pl.BlockSpec((1, S, E), lambda h: (h, 0, 0)),
        ],
        out_specs=[
            pl.BlockSpec((1, S, DH), lambda h: (h, 0, 0)),
            pl.BlockSpec((1, S, DH), lambda h: (h, 0, 0)),
        ],
        out_shape=[
            jax.ShapeDtypeStruct((H, S, DH), F32),
            jax.ShapeDtypeStruct((H, S, DH), F32),
        ],
    )(xh, wv, oh1t, oh2t)


def _attn_body(q_ref, k_ref, v1_ref, v2_ref, m_ref, a1_ref, a2_ref):
    q = q_ref[0]
    k = k_ref[0]
    s = jax.lax.dot_general(q, k, (((1,), (1,)), ((), ())),
                            preferred_element_type=F32)
    s = s * (1.0 / math.sqrt(DH)) + m_ref[...]
    mx = jnp.max(s, axis=-1, keepdims=True)
    p = jnp.exp(s - mx)
    p = p / jnp.sum(p, axis=-1, keepdims=True)

    @pl.when(pl.program_id(1) == 0)
    def _():
        a1_ref[0] = jnp.zeros((S, DH), F32)
        a2_ref[0] = jnp.zeros((S, DH), F32)

    a1_ref[0] += jax.lax.dot_general(p, v1_ref[0], (((0,), (0,)), ((), ())),
                                     preferred_element_type=F32)
    a2_ref[0] += jax.lax.dot_general(p, v2_ref[0], (((0,), (0,)), ((), ())),
                                     preferred_element_type=F32)


def _attn(qh, kh, v1, v2, mask):
    return pl.pallas_call(
        _attn_body,
        grid=(H, NRB),
        in_specs=[
            pl.BlockSpec((1, RB, DH), lambda h, i: (h, i, 0)),
            pl.BlockSpec((1, S, DH), lambda h, i: (h, 0, 0)),
            pl.BlockSpec((1, RB, DH), lambda h, i: (h, i, 0)),
            pl.BlockSpec((1, RB, DH), lambda h, i: (h, i, 0)),
            pl.BlockSpec((RB, S), lambda h, i: (i, 0)),
        ],
        out_specs=[
            pl.BlockSpec((1, S, DH), lambda h, i: (h, 0, 0)),
            pl.BlockSpec((1, S, DH), lambda h, i: (h, 0, 0)),
        ],
        out_shape=[
            jax.ShapeDtypeStruct((H, S, DH), F32),
            jax.ShapeDtypeStruct((H, S, DH), F32),
        ],
    )(qh, kh, v1, v2, mask)


def _oproj_body(a1_ref, a2_ref, wo_ref, php1_ref, php2_ref, o_ref):
    a1 = a1_ref[0]
    a2 = a2_ref[0]
    php1 = php1_ref[0]
    php2 = php2_ref[0]
    acc = jnp.zeros((S, DH), F32)
    for e in range(E):
        c = php1[:, e:e + 1] * a1 + php2[:, e:e + 1] * a2
        acc = acc + jnp.dot(c, wo_ref[0, e], preferred_element_type=F32)
    o_ref[0] = acc


def _oproj(a1, a2, wo, php1t, php2t):
    return pl.pallas_call(
        _oproj_body,
        grid=(H,),
        in_specs=[
            pl.BlockSpec((1, S, DH), lambda h: (h, 0, 0)),
            pl.BlockSpec((1, S, DH), lambda h: (h, 0, 0)),
            pl.BlockSpec((1, E, DH, DH), lambda h: (h, 0, 0, 0)),
            pl.BlockSpec((1, S, E), lambda h: (h, 0, 0)),
            pl.BlockSpec((1, S, E), lambda h: (h, 0, 0)),
        ],
        out_specs=pl.BlockSpec((1, S, DH), lambda h: (h, 0, 0)),
        out_shape=jax.ShapeDtypeStruct((H, S, DH), F32),
    )(a1, a2, wo, php1t, php2t)


def _prep2_body(x_ref, ao_ref, s_ref, b_ref, gw_ref,
                xm_ref, xn2_ref, oh1_ref, oh2_ref, php1_ref, php2_ref):
    xm = x_ref[...] + ao_ref[...]
    xm_ref[...] = xm
    xn2 = _ln(xm, s_ref[...], b_ref[...])
    xn2_ref[...] = xn2
    gl = jnp.dot(xn2, gw_ref[...], preferred_element_type=F32)
    oh1, oh2, p1, p2 = _top2(gl)
    oh1_ref[...] = oh1
    oh2_ref[...] = oh2
    php1_ref[...] = oh1 * p1
    php2_ref[...] = oh2 * p2


def _prep2(x, ao, ln_s, ln_b, gw):
    return pl.pallas_call(
        _prep2_body,
        grid=(NRB,),
        in_specs=[
            pl.BlockSpec((RB, D), lambda i: (i, 0)),
            pl.BlockSpec((RB, D), lambda i: (i, 0)),
            pl.BlockSpec((1, D), lambda i: (0, 0)),
            pl.BlockSpec((1, D), lambda i: (0, 0)),
            pl.BlockSpec((D, E), lambda i: (0, 0)),
        ],
        out_specs=[
            pl.BlockSpec((RB, D), lambda i: (i, 0)),
            pl.BlockSpec((RB, D), lambda i: (i, 0)),
            pl.BlockSpec((RB, E), lambda i: (i, 0)),
            pl.BlockSpec((RB, E), lambda i: (i, 0)),
            pl.BlockSpec((RB, E), lambda i: (i, 0)),
            pl.BlockSpec((RB, E), lambda i: (i, 0)),
        ],
        out_shape=[
            jax.ShapeDtypeStruct((S, D), F32),
            jax.ShapeDtypeStruct((S, D), F32),
            jax.ShapeDtypeStruct((S, E), F32),
            jax.ShapeDtypeStruct((S, E), F32),
            jax.ShapeDtypeStruct((S, E), F32),
            jax.ShapeDtypeStruct((S, E), F32),
        ],
    )(x, ao, ln_s, ln_b, gw)


def _cap_body(oh1_ref, oh2_ref, php1_ref, php2_ref,
              w_ref, tok_ref, imp_ref, run_ref):
    i = pl.program_id(0)

    @pl.when(i == 0)
    def _():
        run_ref[...] = jnp.zeros_like(run_ref)
        tok_ref[...] = jnp.zeros_like(tok_ref)
        imp_ref[...] = jnp.zeros_like(imp_ref)

    oh1 = oh1_ref[...]
    oh2 = oh2_ref[...]
    ohsum = oh1 + oh2
    r = jax.lax.broadcasted_iota(jnp.int32, (RB, RB), 0)
    c = jax.lax.broadcasted_iota(jnp.int32, (RB, RB), 1)
    ltri = (r > c).astype(F32)
    before = jnp.dot(ltri, ohsum, preferred_element_type=F32) + run_ref[...]
    pos0 = before + 1.0
    pos1 = before + oh1 + 1.0
    keep0 = (pos0 <= CAP).astype(F32)
    keep1 = (pos1 <= CAP).astype(F32)
    pr0 = php1_ref[...] * keep0
    pr1 = php2_ref[...] * keep1
    denom = jnp.sum(pr0 + pr1, axis=-1, keepdims=True) + 1e-9
    pr0 = pr0 / denom
    pr1 = pr1 / denom
    w_ref[...] = pr0 + pr1
    m0 = jnp.where(pr0 > 0, oh1, 0.0)
    m1 = jnp.where(pr1 > 0, oh2, 0.0)
    tok_ref[...] += jnp.sum(m0 + m1, axis=0, keepdims=True)
    imp_ref[...] += jnp.sum(pr0 + pr1, axis=0, keepdims=True)
    run_ref[...] += jnp.sum(ohsum, axis=0, keepdims=True)


def _capacity(oh1, oh2, php1, php2):
    return pl.pallas_call(
        _cap_body,
        grid=(NRB,),
        in_specs=[
            pl.BlockSpec((RB, E), lambda i: (i, 0)),
            pl.BlockSpec((RB, E), lambda i: (i, 0)),
            pl.BlockSpec((RB, E), lambda i: (i, 0)),
            pl.BlockSpec((RB, E), lambda i: (i, 0)),
        ],
        out_specs=[
            pl.BlockSpec((RB, E), lambda i: (i, 0)),
            pl.BlockSpec((1, E), lambda i: (0, 0)),
            pl.BlockSpec((1, E), lambda i: (0, 0)),
        ],
        out_shape=[
            jax.ShapeDtypeStruct((S, E), F32),
            jax.ShapeDtypeStruct((1, E), F32),
            jax.ShapeDtypeStruct((1, E), F32),
        ],
        scratch_shapes=[pltpu.VMEM((1, E), F32)],
    )(oh1, oh2, php1, php2)


def _ffn_body(xn2_ref, w_ref, xm_ref, w1_ref, w2_ref, y_ref, acc_ref):
    e = pl.program_id(0)
    i = pl.program_id(1)
    lane = jax.lax.broadcasted_iota(jnp.int32, (RB, E), 1)
    we = jnp.sum(jnp.where(lane == e, w_ref[...], 0.0), axis=-1, keepdims=True)
    xw = xn2_ref[...] * we
    h1 = jnp.maximum(jnp.dot(xw, w1_ref[0], preferred_element_type=F32), 0.0)
    out = jnp.dot(h1, w2_ref[0], preferred_element_type=F32)
    sl = pl.ds(i * RB, RB)

    @pl.when(e == 0)
    def _():
        acc_ref[sl, :] = xm_ref[...]

    acc_ref[sl, :] += out
    y_ref[...] = acc_ref[sl, :]


def _ffn(xn2, w, xm, w1, w2):
    return pl.pallas_call(
        _ffn_body,
        grid=(E, NRB),
        in_specs=[
            pl.BlockSpec((RB, D), lambda e, i: (i, 0)),
            pl.BlockSpec((RB, E), lambda e, i: (i, 0)),
            pl.BlockSpec((RB, D), lambda e, i: (i, 0)),
            pl.BlockSpec((1, D, DFF), lambda e, i: (e, 0, 0)),
            pl.BlockSpec((1, DFF, D), lambda e, i: (e, 0, 0)),
        ],
        out_specs=pl.BlockSpec((RB, D), lambda e, i: (i, 0)),
        out_shape=jax.ShapeDtypeStruct((S, D), F32),
        scratch_shapes=[pltpu.VMEM((S, D), F32)],
    )(xn2, w, xm, w1, w2)


def _aux_body(hist_ref, tok_ref, imp_ref, o_ref):
    ema = 0.01 * hist_ref[...] / (S * K)
    pb = ema / (jnp.sum(ema) + 1e-9)
    aux1 = jnp.sum(pb * pb) * (E * H)
    tok = tok_ref[...]
    imp = imp_ref[...]
    aux2 = jnp.sum((tok / jnp.sum(tok)) * (imp / jnp.sum(imp))) * E
    o_ref[...] = jnp.full((1, 1), aux1 + aux2, F32)


def _aux(hist, tok, imp):
    return pl.pallas_call(
        _aux_body,
        in_specs=[
            pl.BlockSpec((H, E), lambda: (0, 0)),
            pl.BlockSpec((1, E), lambda: (0, 0)),
            pl.BlockSpec((1, E), lambda: (0, 0)),
        ],
        out_specs=pl.BlockSpec((1, 1), lambda: (0, 0)),
        out_shape=jax.ShapeDtypeStruct((1, 1), F32),
    )(hist, tok, imp)


def kernel(x, mask, ln1_scale, ln1_bias, ln2_scale, ln2_bias,
           W_q, W_k, W_v, W_o, router_W, gate_W, W1, W2):
    x2 = x.reshape(S, D)
    xn, q, k, gl = _prep1(x2, ln1_scale.reshape(1, D), ln1_bias.reshape(1, D),
                          W_q, W_k, router_W)
    oh1, oh2, php1, php2, hist = _route(gl.reshape(S, H, E))

    to_h = lambda a: a.reshape(S, H, DH).transpose(1, 0, 2)
    te = lambda a: a.transpose(1, 0, 2)
    v1, v2 = _vproj(to_h(xn), W_v, te(oh1), te(oh2))
    a1, a2 = _attn(to_h(q), to_h(k), v1, v2, mask)
    ao = _oproj(a1, a2, W_o, te(php1), te(php2))
    ao2 = ao.transpose(1, 0, 2).reshape(S, D)

    xm, xn2, goh1, goh2, gphp1, gphp2 = _prep2(
        x2, ao2, ln2_scale.reshape(1, D), ln2_bias.reshape(1, D), gate_W)
    w, tok, imp = _capacity(goh1, goh2, gphp1, gphp2)
    y = _ffn(xn2, w, xm, W1, W2)
    aux = _aux(hist, tok, imp)
    return y.reshape(1, S, D), aux.reshape(())
